# baseline (speedup 1.0000x reference)
import functools

import jax
import jax.numpy as jnp
from jax import lax
from jax.experimental import pallas as pl
from jax.experimental.pallas import tpu as pltpu
from jax.experimental.pallas import tpu_sc as plsc

_N = 10000
_E = 320000
_NC = 2
_NS = 16
_NW = _NC * _NS
_C = 512
_CSUB = _C // 128
_EPW = 10240
_NCHUNK = _EPW // _C
_EPAD = _NW * _EPW
_NP = 10112
_EPS = 1e-5


_CB = 128
_CROWS = _CB // 128
_NCH = _EPW // _CB
_IROWS = _EPW // 128


def _make_spmm(F, ring):
    rows_per_tile = _NP // _NS
    mesh = plsc.VectorSubcoreMesh(
        core_axis_name="c", subcore_axis_name="s",
        num_cores=_NC, num_subcores=_NS)

    @functools.partial(
        pl.kernel,
        out_type=jax.ShapeDtypeStruct((_NC, _NP, F), jnp.float32),
        mesh=mesh,
        scratch_types=[
            pltpu.VMEM((_IROWS, 128), jnp.int32),
            pltpu.VMEM((_IROWS, 128), jnp.int32),
            pltpu.VMEM((_IROWS, 128), jnp.float32),
            [pltpu.VMEM((_CB, F), jnp.float32) for _ in range(ring)],
            pltpu.VMEM_SHARED((_NP, F), jnp.float32),
            [pltpu.SemaphoreType.DMA for _ in range(ring)],
            [pltpu.SemaphoreType.DMA for _ in range(ring)],
        ],
        compiler_params=pltpu.CompilerParams(use_tc_tiling_on_sc=False),
    )
    def spmm(sup_hbm, src_hbm, dst_hbm, ew_hbm, srcp_hbm, dstp_hbm,
             ewp_hbm, zero_hbm, out_hbm,
             srcv, dstv, ewv, rows, acc, gsem, ssem):
        cid = lax.axis_index("c")
        sid = lax.axis_index("s")
        wid = cid * _NS + sid
        tbase = sid * rows_per_tile
        pltpu.sync_copy(zero_hbm.at[pl.ds(tbase, rows_per_tile)],
                        acc.at[pl.ds(tbase, rows_per_tile)])
        @pl.when(wid < _NW - 1)
        def _():
            pltpu.sync_copy(src_hbm.at[pl.ds(wid * _IROWS, _IROWS)], srcv)
            pltpu.sync_copy(dst_hbm.at[pl.ds(wid * _IROWS, _IROWS)], dstv)
            pltpu.sync_copy(ew_hbm.at[pl.ds(wid * _IROWS, _IROWS)], ewv)

        @pl.when(wid == _NW - 1)
        def _():
            base = (_NW - 1) * _IROWS
            pltpu.sync_copy(src_hbm.at[pl.ds(base, 16)], srcv.at[pl.ds(0, 16)])
            pltpu.sync_copy(dst_hbm.at[pl.ds(base, 16)], dstv.at[pl.ds(0, 16)])
            pltpu.sync_copy(ew_hbm.at[pl.ds(base, 16)], ewv.at[pl.ds(0, 16)])
            pltpu.sync_copy(srcp_hbm, srcv.at[pl.ds(16, 64)])
            pltpu.sync_copy(dstp_hbm, dstv.at[pl.ds(16, 64)])
            pltpu.sync_copy(ewp_hbm, ewv.at[pl.ds(16, 64)])
        plsc.subcore_barrier()

        def fire_gather(k, b):
            pltpu.async_copy(sup_hbm.at[srcv.at[k]], rows[b], gsem[b])

        def wait_gather(k, b):
            pltpu.make_async_copy(sup_hbm.at[srcv.at[k]], rows[b],
                                  gsem[b]).wait()

        def fire_scatter(k, b):
            pltpu.async_copy(rows[b], acc.at[dstv.at[k]], ssem[b], add=True)

        def wait_scatter(k, b):
            pltpu.make_async_copy(rows[b], acc.at[dstv.at[k]], ssem[b]).wait()

        lidx = [jnp.full((16, 1), l, jnp.int32) for l in range(16)]
        gdn = lax.GatherDimensionNumbers(
            offset_dims=(), collapsed_slice_dims=(0,), start_index_map=(0,))

        def scale(k, b):
            def g_body(g, c2):
                ew16 = ewv[k, pl.ds(g * 16, 16)]
                for l in range(16):
                    w16 = lax.gather(
                        ew16, lidx[l], gdn, slice_sizes=(1,),
                        mode=lax.GatherScatterMode.PROMISE_IN_BOUNDS)
                    e = g * 16 + l
                    for f in range(F // 16):
                        sl = pl.ds(f * 16, 16)
                        rows[b][e, sl] = rows[b][e, sl] * w16
                return c2
            lax.fori_loop(0, 8, g_body, 0)

        if ring == 4:
            fire_gather(0, 0)
            fire_gather(1, 1)
            for b in range(4):
                k = b
                if k >= 2:
                    wait_scatter(k - 2, (b + 2) % 4)
                fire_gather(k + 2, (b + 2) % 4)
                wait_gather(k, b)
                scale(k, b)
                fire_scatter(k, b)

            def quad(kk, carry):
                for b in range(4):
                    k = 4 * kk + b
                    bx = (b + 2) % 4
                    wait_scatter(k - 2, bx)
                    fire_gather(k + 2, bx)
                    wait_gather(k, b)
                    scale(k, b)
                    fire_scatter(k, b)
                return carry
            lax.fori_loop(1, _NCH // 4 - 1, quad, 0)

            for b in range(4):
                k = _NCH - 4 + b
                bx = (b + 2) % 4
                wait_scatter(k - 2, bx)
                if k + 2 < _NCH:
                    fire_gather(k + 2, bx)
                wait_gather(k, b)
                scale(k, b)
                fire_scatter(k, b)
            wait_scatter(_NCH - 2, 2)
            wait_scatter(_NCH - 1, 3)
        else:
            fire_gather(0, 0)
            fire_gather(1, 1)
            wait_gather(0, 0)
            scale(0, 0)
            fire_scatter(0, 0)

            def pair(kk, carry):
                for (k, b) in ((2 * kk + 1, 1), (2 * kk + 2, 0)):
                    wait_scatter(k - 1, 1 - b)
                    fire_gather(k + 1, 1 - b)
                    wait_gather(k, b)
                    scale(k, b)
                    fire_scatter(k, b)
                return carry
            lax.fori_loop(0, (_NCH - 2) // 2, pair, 0)

            k = _NCH - 1
            wait_scatter(k - 1, 0)
            wait_gather(k, 1)
            scale(k, 1)
            fire_scatter(k, 1)
            wait_scatter(k, 1)

        plsc.subcore_barrier()
        pltpu.sync_copy(acc.at[pl.ds(tbase, rows_per_tile)],
                        out_hbm.at[cid].at[pl.ds(tbase, rows_per_tile)])

    return spmm


_spmm64 = _make_spmm(64, ring=4)
_spmm32 = _make_spmm(32, ring=4)


def _mm_body(x_ref, w_ref, o_ref):
    o_ref[...] = jnp.dot(x_ref[...], w_ref[...],
                         preferred_element_type=jnp.float32)


def _tc_support1(x, w):
    return pl.pallas_call(
        _mm_body,
        grid=(10,),
        in_specs=[pl.BlockSpec((1000, 128), lambda i: (i, 0)),
                  pl.BlockSpec((128, 64), lambda i: (0, 0))],
        out_specs=pl.BlockSpec((1000, 64), lambda i: (i, 0)),
        out_shape=jax.ShapeDtypeStruct((_N, 64), jnp.float32),
    )(x, w)


def _combine_mm_body(p_ref, w_ref, o_ref):
    h = jnp.maximum(p_ref[0] + p_ref[1], 0.0)
    o_ref[...] = jnp.dot(h, w_ref[...], preferred_element_type=jnp.float32)


def _tc_support23(parts, wcat):
    return pl.pallas_call(
        _combine_mm_body,
        grid=(10,),
        in_specs=[pl.BlockSpec((2, 1000, 64), lambda i: (0, i, 0)),
                  pl.BlockSpec((64, 32), lambda i: (0, 0))],
        out_specs=pl.BlockSpec((1000, 32), lambda i: (i, 0)),
        out_shape=jax.ShapeDtypeStruct((_N, 32), jnp.float32),
    )(parts, wcat)


_RB = 400


def _dec_body(pf_ref, pb_ref,
              W1, b1, g1, be1, W2, b2, g2, be2, W3, b3, g3, be3,
              W4, b4, g4, be4, W5, b5,
              dc_ref, mu_ref, lv_ref, xr_ref):
    s_blk = pb_ref[0] + pb_ref[1]
    mu_blk = s_blk[:, :16]
    mu_ref[...] = mu_blk
    lv_ref[...] = s_blk[:, 16:]
    s_full = pf_ref[0] + pf_ref[1]
    mu_full = s_full[:, :16]
    dc_ref[...] = lax.dot_general(
        mu_blk, mu_full, (((1,), (1,)), ((), ())),
        preferred_element_type=jnp.float32)
    inv = 1.0 / jnp.sqrt(1.0 + _EPS)

    def fc(o, W, b, g, be):
        t = (jnp.dot(o, W[...], preferred_element_type=jnp.float32)
             + b[0]) * inv
        return jnp.maximum(g[0] * t + be[0], 0.0)

    o = fc(mu_blk, W1, b1, g1, be1)
    o = fc(o, W2, b2, g2, be2)
    o = fc(o, W3, b3, g3, be3)
    o = fc(o, W4, b4, g4, be4)
    xr_ref[...] = jnp.dot(o, W5[...], preferred_element_type=jnp.float32) + b5[0]


def _tc_decode(parts, W1, b1, g1, be1, W2, b2, g2, be2, W3, b3, g3, be3,
               W4, b4, g4, be4, W5, b5):
    nblk = _N // _RB
    full = lambda shape: pl.BlockSpec(shape, lambda i: tuple(0 for _ in shape))
    return pl.pallas_call(
        _dec_body,
        grid=(nblk,),
        in_specs=[
            pl.BlockSpec((2, _N, 32), lambda i: (0, 0, 0)),
            pl.BlockSpec((2, _RB, 32), lambda i: (0, i, 0)),
            full((16, 64)), full((1, 64)), full((1, 64)), full((1, 64)),
            full((64, 128)), full((1, 128)), full((1, 128)), full((1, 128)),
            full((128, 128)), full((1, 128)), full((1, 128)), full((1, 128)),
            full((128, 64)), full((1, 64)), full((1, 64)), full((1, 64)),
            full((64, 128)), full((1, 128)),
        ],
        out_specs=[
            pl.BlockSpec((_RB, _N), lambda i: (i, 0)),
            pl.BlockSpec((_RB, 16), lambda i: (i, 0)),
            pl.BlockSpec((_RB, 16), lambda i: (i, 0)),
            pl.BlockSpec((_RB, 128), lambda i: (i, 0)),
        ],
        out_shape=[
            jax.ShapeDtypeStruct((_N, _N), jnp.float32),
            jax.ShapeDtypeStruct((_N, 16), jnp.float32),
            jax.ShapeDtypeStruct((_N, 16), jnp.float32),
            jax.ShapeDtypeStruct((_N, 128), jnp.float32),
        ],
    )(parts, parts, W1, b1, g1, be1, W2, b2, g2, be2, W3, b3, g3, be3,
      W4, b4, g4, be4, W5, b5)


def kernel(x, edge_index, edge_weight, Wg1, Wg2, Wg3, W1, b1, W2, b2,
           W3, b3, W4, b4, W5, b5, g1, be1, g2, be2, g3, be3, g4, be4):
    dst = edge_index[0]
    src = edge_index[1]
    src2d = src.reshape(_E // 128, 128)
    dst2d = dst.reshape(_E // 128, 128)
    ew2d = edge_weight.reshape(_E // 128, 128)
    padidx = (jnp.arange(60 * 128, dtype=jnp.int32) % _N).reshape(60, 128)
    srcp = jnp.concatenate([src2d[-4:], padidx], axis=0)
    dstp = jnp.concatenate([dst2d[-4:], padidx], axis=0)
    ewp = jnp.concatenate(
        [ew2d[-4:], jnp.zeros((60, 128), jnp.float32)], axis=0)
    zeros64 = jnp.zeros((_NP, 64), jnp.float32)
    zeros32 = jnp.zeros((_NP, 32), jnp.float32)

    support1 = _tc_support1(x, Wg1)
    parts1 = _spmm64(support1, src2d, dst2d, ew2d, srcp, dstp, ewp, zeros64)
    wcat = jnp.concatenate([Wg2, Wg3], axis=1)
    s23 = _tc_support23(parts1, wcat)
    parts23 = _spmm32(s23, src2d, dst2d, ew2d, srcp, dstp, ewp, zeros32)

    r2 = lambda v: v.reshape(1, -1)
    dc, mu, logvar, xr = _tc_decode(
        parts23, W1, r2(b1), r2(g1), r2(be1), W2, r2(b2), r2(g2), r2(be2),
        W3, r2(b3), r2(g3), r2(be3), W4, r2(b4), r2(g4), r2(be4), W5, r2(b5))
    return (dc, mu, logvar, mu, xr)


# speedup vs baseline: 1.2803x; 1.2803x over previous
import functools

import jax
import jax.numpy as jnp
from jax import lax
from jax.experimental import pallas as pl
from jax.experimental.pallas import tpu as pltpu
from jax.experimental.pallas import tpu_sc as plsc

_N = 10000
_E = 320000
_NC = 2
_NS = 16
_NW = _NC * _NS
_C = 512
_CSUB = _C // 128
_EPW = 10240
_NCHUNK = _EPW // _C
_EPAD = _NW * _EPW
_NP = 10112
_EPS = 1e-5


_CB = 128
_CROWS = _CB // 128
_NCH = _EPW // _CB
_IROWS = _EPW // 128


def _make_spmm(F):
    rows_per_tile = _NP // _NS
    mesh = plsc.VectorSubcoreMesh(
        core_axis_name="c", subcore_axis_name="s",
        num_cores=_NC, num_subcores=_NS)

    @functools.partial(
        pl.kernel,
        out_type=jax.ShapeDtypeStruct((_NC, _NP, F), jnp.float32),
        mesh=mesh,
        scratch_types=[
            pltpu.VMEM((_IROWS, 128), jnp.int32),
            pltpu.VMEM((_IROWS, 128), jnp.int32),
            pltpu.VMEM((_IROWS, 128), jnp.float32),
            [pltpu.VMEM((_CB, F), jnp.float32) for _ in range(3)],
            [pltpu.VMEM((_CB, F), jnp.float32) for _ in range(3)],
            pltpu.VMEM_SHARED((_NP, F), jnp.float32),
            [pltpu.SemaphoreType.DMA for _ in range(3)],
            [pltpu.SemaphoreType.DMA for _ in range(3)],
        ],
        compiler_params=pltpu.CompilerParams(use_tc_tiling_on_sc=False),
    )
    def spmm(sup_hbm, src_hbm, dst_hbm, ew_hbm, srcp_hbm, dstp_hbm,
             ewp_hbm, zero_hbm, out_hbm,
             srcv, dstv, ewv, rin, rout, acc, gsem, ssem):
        cid = lax.axis_index("c")
        sid = lax.axis_index("s")
        wid = cid * _NS + sid
        tbase = sid * rows_per_tile
        pltpu.sync_copy(zero_hbm.at[pl.ds(tbase, rows_per_tile)],
                        acc.at[pl.ds(tbase, rows_per_tile)])
        @pl.when(wid < _NW - 1)
        def _():
            pltpu.sync_copy(src_hbm.at[pl.ds(wid * _IROWS, _IROWS)], srcv)
            pltpu.sync_copy(dst_hbm.at[pl.ds(wid * _IROWS, _IROWS)], dstv)
            pltpu.sync_copy(ew_hbm.at[pl.ds(wid * _IROWS, _IROWS)], ewv)

        @pl.when(wid == _NW - 1)
        def _():
            base = (_NW - 1) * _IROWS
            pltpu.sync_copy(src_hbm.at[pl.ds(base, 16)], srcv.at[pl.ds(0, 16)])
            pltpu.sync_copy(dst_hbm.at[pl.ds(base, 16)], dstv.at[pl.ds(0, 16)])
            pltpu.sync_copy(ew_hbm.at[pl.ds(base, 16)], ewv.at[pl.ds(0, 16)])
            pltpu.sync_copy(srcp_hbm, srcv.at[pl.ds(16, 64)])
            pltpu.sync_copy(dstp_hbm, dstv.at[pl.ds(16, 64)])
            pltpu.sync_copy(ewp_hbm, ewv.at[pl.ds(16, 64)])
        plsc.subcore_barrier()

        def fire_gather(k, b):
            pltpu.async_copy(sup_hbm.at[srcv.at[k]], rin[b], gsem[b])

        def wait_gather(k, b):
            pltpu.make_async_copy(sup_hbm.at[srcv.at[k]], rin[b],
                                  gsem[b]).wait()

        def fire_scatter(k, b):
            pltpu.async_copy(rout[b], acc.at[dstv.at[k]], ssem[b], add=True)

        def wait_scatter(k, b):
            pltpu.make_async_copy(rout[b], acc.at[dstv.at[k]],
                                  ssem[b]).wait()

        def scale(k, b):
            def g_body(g, c2):
                ew16 = ewv[k, pl.ds(g * 16, 16)]
                for l in range(16):
                    w16 = jnp.full((16,), ew16[l], jnp.float32)
                    e = g * 16 + l
                    for f in range(F // 16):
                        sl = pl.ds(f * 16, 16)
                        rout[b][e, sl] = rin[b][e, sl] * w16
                return c2
            lax.fori_loop(0, 8, g_body, 0)

        def position(k, b):
            if isinstance(k, int):
                if k >= 3:
                    wait_scatter(k - 3, b)
            else:
                wait_scatter(k - 3, b)
            wait_gather(k, b)
            scale(k, b)
            fire_scatter(k, b)
            if isinstance(k, int):
                if k + 2 < _NCH:
                    fire_gather(k + 2, (b + 2) % 3)
            else:
                fire_gather(k + 2, (b + 2) % 3)

        fire_gather(0, 0)
        fire_gather(1, 1)
        for k in range(3):
            position(k, k)

        def triple(tt, carry):
            for b0 in range(3):
                position(3 * tt + b0, b0)
            return carry
        lax.fori_loop(1, (_NCH - 2) // 3, triple, 0)

        for k in range(_NCH - 2, _NCH):
            position(k, k % 3)
        for k in range(_NCH - 3, _NCH):
            wait_scatter(k, k % 3)

        plsc.subcore_barrier()
        pltpu.sync_copy(acc.at[pl.ds(tbase, rows_per_tile)],
                        out_hbm.at[cid].at[pl.ds(tbase, rows_per_tile)])

    return spmm


_spmm64 = _make_spmm(64)
_spmm32 = _make_spmm(32)


def _mm_body(x_ref, w_ref, o_ref):
    o_ref[...] = jnp.dot(x_ref[...], w_ref[...],
                         preferred_element_type=jnp.float32)


def _tc_support1(x, w):
    return pl.pallas_call(
        _mm_body,
        grid=(10,),
        in_specs=[pl.BlockSpec((1000, 128), lambda i: (i, 0)),
                  pl.BlockSpec((128, 64), lambda i: (0, 0))],
        out_specs=pl.BlockSpec((1000, 64), lambda i: (i, 0)),
        out_shape=jax.ShapeDtypeStruct((_N, 64), jnp.float32),
    )(x, w)


def _combine_mm_body(p_ref, w_ref, o_ref):
    h = jnp.maximum(p_ref[0] + p_ref[1], 0.0)
    o_ref[...] = jnp.dot(h, w_ref[...], preferred_element_type=jnp.float32)


def _tc_support23(parts, wcat):
    return pl.pallas_call(
        _combine_mm_body,
        grid=(10,),
        in_specs=[pl.BlockSpec((2, 1000, 64), lambda i: (0, i, 0)),
                  pl.BlockSpec((64, 32), lambda i: (0, 0))],
        out_specs=pl.BlockSpec((1000, 32), lambda i: (i, 0)),
        out_shape=jax.ShapeDtypeStruct((_N, 32), jnp.float32),
    )(parts, wcat)


_RB = 400


def _dec_body(pf_ref, pb_ref,
              W1, b1, g1, be1, W2, b2, g2, be2, W3, b3, g3, be3,
              W4, b4, g4, be4, W5, b5,
              dc_ref, mu_ref, lv_ref, xr_ref):
    s_blk = pb_ref[0] + pb_ref[1]
    mu_blk = s_blk[:, :16]
    mu_ref[...] = mu_blk
    lv_ref[...] = s_blk[:, 16:]
    s_full = pf_ref[0] + pf_ref[1]
    mu_full = s_full[:, :16]
    dc_ref[...] = lax.dot_general(
        mu_blk, mu_full, (((1,), (1,)), ((), ())),
        preferred_element_type=jnp.float32)
    inv = 1.0 / jnp.sqrt(1.0 + _EPS)

    def fc(o, W, b, g, be):
        t = (jnp.dot(o, W[...], preferred_element_type=jnp.float32)
             + b[0]) * inv
        return jnp.maximum(g[0] * t + be[0], 0.0)

    o = fc(mu_blk, W1, b1, g1, be1)
    o = fc(o, W2, b2, g2, be2)
    o = fc(o, W3, b3, g3, be3)
    o = fc(o, W4, b4, g4, be4)
    xr_ref[...] = jnp.dot(o, W5[...], preferred_element_type=jnp.float32) + b5[0]


def _tc_decode(parts, W1, b1, g1, be1, W2, b2, g2, be2, W3, b3, g3, be3,
               W4, b4, g4, be4, W5, b5):
    nblk = _N // _RB
    full = lambda shape: pl.BlockSpec(shape, lambda i: tuple(0 for _ in shape))
    return pl.pallas_call(
        _dec_body,
        grid=(nblk,),
        in_specs=[
            pl.BlockSpec((2, _N, 32), lambda i: (0, 0, 0)),
            pl.BlockSpec((2, _RB, 32), lambda i: (0, i, 0)),
            full((16, 64)), full((1, 64)), full((1, 64)), full((1, 64)),
            full((64, 128)), full((1, 128)), full((1, 128)), full((1, 128)),
            full((128, 128)), full((1, 128)), full((1, 128)), full((1, 128)),
            full((128, 64)), full((1, 64)), full((1, 64)), full((1, 64)),
            full((64, 128)), full((1, 128)),
        ],
        out_specs=[
            pl.BlockSpec((_RB, _N), lambda i: (i, 0)),
            pl.BlockSpec((_RB, 16), lambda i: (i, 0)),
            pl.BlockSpec((_RB, 16), lambda i: (i, 0)),
            pl.BlockSpec((_RB, 128), lambda i: (i, 0)),
        ],
        out_shape=[
            jax.ShapeDtypeStruct((_N, _N), jnp.float32),
            jax.ShapeDtypeStruct((_N, 16), jnp.float32),
            jax.ShapeDtypeStruct((_N, 16), jnp.float32),
            jax.ShapeDtypeStruct((_N, 128), jnp.float32),
        ],
    )(parts, parts, W1, b1, g1, be1, W2, b2, g2, be2, W3, b3, g3, be3,
      W4, b4, g4, be4, W5, b5)


def kernel(x, edge_index, edge_weight, Wg1, Wg2, Wg3, W1, b1, W2, b2,
           W3, b3, W4, b4, W5, b5, g1, be1, g2, be2, g3, be3, g4, be4):
    dst = edge_index[0]
    src = edge_index[1]
    src2d = src.reshape(_E // 128, 128)
    dst2d = dst.reshape(_E // 128, 128)
    ew2d = edge_weight.reshape(_E // 128, 128)
    padidx = (jnp.arange(60 * 128, dtype=jnp.int32) % _N).reshape(60, 128)
    srcp = jnp.concatenate([src2d[-4:], padidx], axis=0)
    dstp = jnp.concatenate([dst2d[-4:], padidx], axis=0)
    ewp = jnp.concatenate(
        [ew2d[-4:], jnp.zeros((60, 128), jnp.float32)], axis=0)
    zeros64 = jnp.zeros((_NP, 64), jnp.float32)
    zeros32 = jnp.zeros((_NP, 32), jnp.float32)

    support1 = _tc_support1(x, Wg1)
    parts1 = _spmm64(support1, src2d, dst2d, ew2d, srcp, dstp, ewp, zeros64)
    wcat = jnp.concatenate([Wg2, Wg3], axis=1)
    s23 = _tc_support23(parts1, wcat)
    parts23 = _spmm32(s23, src2d, dst2d, ew2d, srcp, dstp, ewp, zeros32)

    r2 = lambda v: v.reshape(1, -1)
    dc, mu, logvar, xr = _tc_decode(
        parts23, W1, r2(b1), r2(g1), r2(be1), W2, r2(b2), r2(g2), r2(be2),
        W3, r2(b3), r2(g3), r2(be3), W4, r2(b4), r2(g4), r2(be4), W5, r2(b5))
    return (dc, mu, logvar, mu, xr)


# speedup vs baseline: 1.3338x; 1.0418x over previous
import functools

import jax
import jax.numpy as jnp
from jax import lax
from jax.experimental import pallas as pl
from jax.experimental.pallas import tpu as pltpu
from jax.experimental.pallas import tpu_sc as plsc

_N = 10000
_E = 320000
_NC = 2
_NS = 16
_NW = _NC * _NS
_C = 512
_CSUB = _C // 128
_EPW = 10240
_NCHUNK = _EPW // _C
_EPAD = _NW * _EPW
_NP = 10112
_EPS = 1e-5


_CB = 128
_CROWS = _CB // 128
_NCH = _EPW // _CB
_IROWS = _EPW // 128


def _make_spmm(F):
    rows_per_tile = _NP // _NS
    mesh = plsc.VectorSubcoreMesh(
        core_axis_name="c", subcore_axis_name="s",
        num_cores=_NC, num_subcores=_NS)

    @functools.partial(
        pl.kernel,
        out_type=jax.ShapeDtypeStruct((_NC, _NP, F), jnp.float32),
        mesh=mesh,
        scratch_types=[
            pltpu.VMEM((_IROWS, 128), jnp.int32),
            pltpu.VMEM((_IROWS, 128), jnp.int32),
            pltpu.VMEM((_IROWS, 128), jnp.float32),
            [pltpu.VMEM((_CB, F), jnp.float32) for _ in range(3)],
            [pltpu.VMEM((_CB, F), jnp.float32) for _ in range(3)],
            pltpu.VMEM_SHARED((_NP, F), jnp.float32),
            [pltpu.SemaphoreType.DMA for _ in range(3)],
            [pltpu.SemaphoreType.DMA for _ in range(3)],
        ],
        compiler_params=pltpu.CompilerParams(use_tc_tiling_on_sc=False),
    )
    def spmm(sup_hbm, src_hbm, dst_hbm, ew_hbm, srcp_hbm, dstp_hbm,
             ewp_hbm, zero_hbm, out_hbm,
             srcv, dstv, ewv, rin, rout, acc, gsem, ssem):
        cid = lax.axis_index("c")
        sid = lax.axis_index("s")
        wid = cid * _NS + sid
        tbase = sid * rows_per_tile
        pltpu.sync_copy(zero_hbm.at[pl.ds(tbase, rows_per_tile)],
                        acc.at[pl.ds(tbase, rows_per_tile)])
        @pl.when(wid < _NW - 1)
        def _():
            pltpu.sync_copy(src_hbm.at[pl.ds(wid * _IROWS, _IROWS)], srcv)
            pltpu.sync_copy(dst_hbm.at[pl.ds(wid * _IROWS, _IROWS)], dstv)
            pltpu.sync_copy(ew_hbm.at[pl.ds(wid * _IROWS, _IROWS)], ewv)

        @pl.when(wid == _NW - 1)
        def _():
            base = (_NW - 1) * _IROWS
            pltpu.sync_copy(src_hbm.at[pl.ds(base, 16)], srcv.at[pl.ds(0, 16)])
            pltpu.sync_copy(dst_hbm.at[pl.ds(base, 16)], dstv.at[pl.ds(0, 16)])
            pltpu.sync_copy(ew_hbm.at[pl.ds(base, 16)], ewv.at[pl.ds(0, 16)])
            pltpu.sync_copy(srcp_hbm, srcv.at[pl.ds(16, 64)])
            pltpu.sync_copy(dstp_hbm, dstv.at[pl.ds(16, 64)])
            pltpu.sync_copy(ewp_hbm, ewv.at[pl.ds(16, 64)])
        plsc.subcore_barrier()

        def fire_gather(k, b):
            pltpu.async_copy(sup_hbm.at[srcv.at[k]], rin[b], gsem[b])

        def wait_gather(k, b):
            pltpu.make_async_copy(sup_hbm.at[srcv.at[k]], rin[b],
                                  gsem[b]).wait()

        def fire_scatter(k, b):
            pltpu.async_copy(rout[b], acc.at[dstv.at[k]], ssem[b], add=True)

        def wait_scatter(k, b):
            pltpu.make_async_copy(rout[b], acc.at[dstv.at[k]],
                                  ssem[b]).wait()

        def scale(k, b):
            def g_body(g, c2):
                ew16 = ewv[k, pl.ds(g * 16, 16)]
                for l in range(16):
                    w16 = jnp.full((16,), ew16[l], jnp.float32)
                    e = g * 16 + l
                    for f in range(F // 16):
                        sl = pl.ds(f * 16, 16)
                        rout[b][e, sl] = rin[b][e, sl] * w16
                return c2
            lax.fori_loop(0, 8, g_body, 0)

        def position(k, b):
            if isinstance(k, int):
                if k >= 3:
                    wait_scatter(k - 3, b)
            else:
                wait_scatter(k - 3, b)
            if isinstance(k, int):
                if k + 2 < _NCH:
                    fire_gather(k + 2, (b + 2) % 3)
            else:
                fire_gather(k + 2, (b + 2) % 3)
            wait_gather(k, b)
            scale(k, b)
            fire_scatter(k, b)

        fire_gather(0, 0)
        fire_gather(1, 1)
        for k in range(3):
            position(k, k)

        def triple(tt, carry):
            for b0 in range(3):
                position(3 * tt + b0, b0)
            return carry
        lax.fori_loop(1, (_NCH - 2) // 3, triple, 0)

        for k in range(_NCH - 2, _NCH):
            position(k, k % 3)
        for k in range(_NCH - 3, _NCH):
            wait_scatter(k, k % 3)

        plsc.subcore_barrier()
        pltpu.sync_copy(acc.at[pl.ds(tbase, rows_per_tile)],
                        out_hbm.at[cid].at[pl.ds(tbase, rows_per_tile)])

    return spmm


_spmm64 = _make_spmm(64)
_spmm32 = _make_spmm(32)


def _mm_body(x_ref, w_ref, o_ref):
    o_ref[...] = jnp.dot(x_ref[...], w_ref[...],
                         preferred_element_type=jnp.float32)


def _tc_support1(x, w):
    return pl.pallas_call(
        _mm_body,
        grid=(10,),
        in_specs=[pl.BlockSpec((1000, 128), lambda i: (i, 0)),
                  pl.BlockSpec((128, 64), lambda i: (0, 0))],
        out_specs=pl.BlockSpec((1000, 64), lambda i: (i, 0)),
        out_shape=jax.ShapeDtypeStruct((_N, 64), jnp.float32),
    )(x, w)


def _combine_mm_body(p_ref, w_ref, o_ref):
    h = jnp.maximum(p_ref[0] + p_ref[1], 0.0)
    o_ref[...] = jnp.dot(h, w_ref[...], preferred_element_type=jnp.float32)


def _tc_support23(parts, wcat):
    return pl.pallas_call(
        _combine_mm_body,
        grid=(10,),
        in_specs=[pl.BlockSpec((2, 1000, 64), lambda i: (0, i, 0)),
                  pl.BlockSpec((64, 32), lambda i: (0, 0))],
        out_specs=pl.BlockSpec((1000, 32), lambda i: (i, 0)),
        out_shape=jax.ShapeDtypeStruct((_N, 32), jnp.float32),
    )(parts, wcat)


_RB = 400


def _dec_body(pf_ref, pb_ref,
              W1, b1, g1, be1, W2, b2, g2, be2, W3, b3, g3, be3,
              W4, b4, g4, be4, W5, b5,
              dc_ref, mu_ref, lv_ref, z_ref, xr_ref):
    s_blk = pb_ref[0] + pb_ref[1]
    mu_blk = s_blk[:, :16]
    mu_ref[...] = mu_blk
    z_ref[...] = mu_blk
    lv_ref[...] = s_blk[:, 16:]
    s_full = pf_ref[0] + pf_ref[1]
    mu_full = s_full[:, :16]
    dc_ref[...] = lax.dot_general(
        mu_blk, mu_full, (((1,), (1,)), ((), ())),
        preferred_element_type=jnp.float32)
    inv = 1.0 / jnp.sqrt(1.0 + _EPS)

    def fc(o, W, b, g, be):
        t = (jnp.dot(o, W[...], preferred_element_type=jnp.float32)
             + b[0]) * inv
        return jnp.maximum(g[0] * t + be[0], 0.0)

    o = fc(mu_blk, W1, b1, g1, be1)
    o = fc(o, W2, b2, g2, be2)
    o = fc(o, W3, b3, g3, be3)
    o = fc(o, W4, b4, g4, be4)
    xr_ref[...] = jnp.dot(o, W5[...], preferred_element_type=jnp.float32) + b5[0]


def _tc_decode(parts, W1, b1, g1, be1, W2, b2, g2, be2, W3, b3, g3, be3,
               W4, b4, g4, be4, W5, b5):
    nblk = _N // _RB
    full = lambda shape: pl.BlockSpec(shape, lambda i: tuple(0 for _ in shape))
    return pl.pallas_call(
        _dec_body,
        grid=(nblk,),
        in_specs=[
            pl.BlockSpec((2, _N, 32), lambda i: (0, 0, 0)),
            pl.BlockSpec((2, _RB, 32), lambda i: (0, i, 0)),
            full((16, 64)), full((1, 64)), full((1, 64)), full((1, 64)),
            full((64, 128)), full((1, 128)), full((1, 128)), full((1, 128)),
            full((128, 128)), full((1, 128)), full((1, 128)), full((1, 128)),
            full((128, 64)), full((1, 64)), full((1, 64)), full((1, 64)),
            full((64, 128)), full((1, 128)),
        ],
        out_specs=[
            pl.BlockSpec((_RB, _N), lambda i: (i, 0)),
            pl.BlockSpec((_RB, 16), lambda i: (i, 0)),
            pl.BlockSpec((_RB, 16), lambda i: (i, 0)),
            pl.BlockSpec((_RB, 16), lambda i: (i, 0)),
            pl.BlockSpec((_RB, 128), lambda i: (i, 0)),
        ],
        out_shape=[
            jax.ShapeDtypeStruct((_N, _N), jnp.float32),
            jax.ShapeDtypeStruct((_N, 16), jnp.float32),
            jax.ShapeDtypeStruct((_N, 16), jnp.float32),
            jax.ShapeDtypeStruct((_N, 16), jnp.float32),
            jax.ShapeDtypeStruct((_N, 128), jnp.float32),
        ],
    )(parts, parts, W1, b1, g1, be1, W2, b2, g2, be2, W3, b3, g3, be3,
      W4, b4, g4, be4, W5, b5)


def kernel(x, edge_index, edge_weight, Wg1, Wg2, Wg3, W1, b1, W2, b2,
           W3, b3, W4, b4, W5, b5, g1, be1, g2, be2, g3, be3, g4, be4):
    dst = edge_index[0]
    src = edge_index[1]
    src2d = src.reshape(_E // 128, 128)
    dst2d = dst.reshape(_E // 128, 128)
    ew2d = edge_weight.reshape(_E // 128, 128)
    padidx = (jnp.arange(60 * 128, dtype=jnp.int32) % _N).reshape(60, 128)
    srcp = jnp.concatenate([src2d[-4:], padidx], axis=0)
    dstp = jnp.concatenate([dst2d[-4:], padidx], axis=0)
    ewp = jnp.concatenate(
        [ew2d[-4:], jnp.zeros((60, 128), jnp.float32)], axis=0)
    zeros64 = jnp.zeros((_NP, 64), jnp.float32)
    zeros32 = jnp.zeros((_NP, 32), jnp.float32)

    support1 = _tc_support1(x, Wg1)
    parts1 = _spmm64(support1, src2d, dst2d, ew2d, srcp, dstp, ewp, zeros64)
    wcat = jnp.concatenate([Wg2, Wg3], axis=1)
    s23 = _tc_support23(parts1, wcat)
    parts23 = _spmm32(s23, src2d, dst2d, ew2d, srcp, dstp, ewp, zeros32)

    r2 = lambda v: v.reshape(1, -1)
    dc, mu, logvar, z, xr = _tc_decode(
        parts23, W1, r2(b1), r2(g1), r2(be1), W2, r2(b2), r2(g2), r2(be2),
        W3, r2(b3), r2(g3), r2(be3), W4, r2(b4), r2(g4), r2(be4), W5, r2(b5))
    return (dc, mu, logvar, z, xr)
